# initial kernel scaffold (unmeasured)
import jax
import jax.numpy as jnp
from jax import lax
from jax.experimental import pallas as pl
from jax.experimental.pallas import tpu as pltpu

N_DEV = 16


def kernel(x, w_mat):
    m_per, k = x.shape
    _, n = w_mat.shape
    n_per = n // N_DEV

    def body(x_ref, w_hbm, out_ref, w_buf, y_buf, w_sems, send_sems, recv_sems):
        my_i = lax.axis_index("i")

        def start_w_dma(s):
            dst = lax.rem(my_i + s, N_DEV)
            dma = pltpu.make_async_copy(
                w_hbm.at[:, pl.ds(dst * n_per, n_per)],
                w_buf.at[s % 2],
                w_sems.at[s % 2],
            )
            dma.start()
            return dma

        w_dmas = {0: start_w_dma(0)}
        pending_send = [None, None]
        for s in range(N_DEV):
            slot = s % 2
            if s + 1 < N_DEV:
                w_dmas[s + 1] = start_w_dma(s + 1)
            w_dmas.pop(s).wait()
            y = jnp.dot(x_ref[...], w_buf[slot], preferred_element_type=jnp.float32)
            if s == 0:
                out_ref[pl.ds(my_i * m_per, m_per), :] = y
            else:
                if pending_send[slot] is not None:
                    pending_send[slot].wait_send()
                y_buf[slot] = y
                dst = lax.rem(my_i + s, N_DEV)
                rdma = pltpu.make_async_remote_copy(
                    src_ref=y_buf.at[slot],
                    dst_ref=out_ref.at[pl.ds(my_i * m_per, m_per), :],
                    send_sem=send_sems.at[slot],
                    recv_sem=recv_sems.at[my_i],
                    device_id=(dst,),
                    device_id_type=pl.DeviceIdType.MESH,
                )
                rdma.start()
                pending_send[slot] = rdma
        for r in pending_send:
            if r is not None:
                r.wait_send()
        for s in range(1, N_DEV):
            src = lax.rem(my_i - s + N_DEV, N_DEV)
            recv = pltpu.make_async_remote_copy(
                src_ref=y_buf.at[0],
                dst_ref=out_ref.at[pl.ds(src * m_per, m_per), :],
                send_sem=send_sems.at[0],
                recv_sem=recv_sems.at[src],
                device_id=(my_i,),
                device_id_type=pl.DeviceIdType.MESH,
            )
            recv.wait_recv()

    return pl.pallas_call(
        body,
        out_shape=jax.ShapeDtypeStruct((N_DEV * m_per, n_per), jnp.float32),
        in_specs=[
            pl.BlockSpec(memory_space=pltpu.VMEM),
            pl.BlockSpec(memory_space=pltpu.ANY),
        ],
        out_specs=pl.BlockSpec(memory_space=pltpu.VMEM),
        scratch_shapes=[
            pltpu.VMEM((2, k, n_per), w_mat.dtype),
            pltpu.VMEM((2, m_per, n_per), jnp.float32),
            pltpu.SemaphoreType.DMA((2,)),
            pltpu.SemaphoreType.DMA((2,)),
            pltpu.SemaphoreType.DMA((N_DEV,)),
        ],
    )(x, w_mat)


# baseline (device time: 162902 ns/iter reference)
import jax
import jax.numpy as jnp
from jax import lax
from jax.experimental import pallas as pl
from jax.experimental.pallas import tpu as pltpu

N_DEV = 16


def kernel(x, w_mat):
    m_per, k = x.shape
    _, n = w_mat.shape
    n_per = n // N_DEV

    def body(x_ref, w_hbm, out_ref, w_buf, y_buf, w_sems, send_sems, recv_sems):
        my_i = lax.axis_index("i")

        def start_w_dma(s):
            dst = lax.rem(my_i + s, N_DEV)
            dma = pltpu.make_async_copy(
                w_hbm.at[:, pl.ds(dst * n_per, n_per)],
                w_buf.at[s % 2],
                w_sems.at[s % 2],
            )
            dma.start()
            return dma

        w_dmas = {0: start_w_dma(0)}
        pending_send = [None, None]
        for s in range(N_DEV):
            slot = s % 2
            if s + 1 < N_DEV:
                w_dmas[s + 1] = start_w_dma(s + 1)
            w_dmas.pop(s).wait()
            y = jnp.dot(x_ref[...], w_buf[slot], preferred_element_type=jnp.float32)
            if s == 0:
                out_ref[pl.ds(my_i * m_per, m_per), :] = y
            else:
                if pending_send[slot] is not None:
                    pending_send[slot].wait_send()
                y_buf[slot] = y
                dst = lax.rem(my_i + s, N_DEV)
                rdma = pltpu.make_async_remote_copy(
                    src_ref=y_buf.at[slot],
                    dst_ref=out_ref.at[pl.ds(my_i * m_per, m_per), :],
                    send_sem=send_sems.at[slot],
                    recv_sem=recv_sems.at[my_i],
                    device_id=(dst,),
                    device_id_type=pl.DeviceIdType.MESH,
                )
                rdma.start()
                pending_send[slot] = rdma
        for r in pending_send:
            if r is not None:
                r.wait_send()
        for s in range(1, N_DEV):
            src = lax.rem(my_i - s + N_DEV, N_DEV)
            recv = pltpu.make_async_remote_copy(
                src_ref=y_buf.at[0],
                dst_ref=out_ref.at[pl.ds(src * m_per, m_per), :],
                send_sem=send_sems.at[0],
                recv_sem=recv_sems.at[src],
                device_id=(my_i,),
                device_id_type=pl.DeviceIdType.MESH,
            )
            recv.wait_recv()

    return pl.pallas_call(
        body,
        out_shape=jax.ShapeDtypeStruct((N_DEV * m_per, n_per), jnp.float32),
        in_specs=[
            pl.BlockSpec(memory_space=pltpu.VMEM),
            pl.BlockSpec(memory_space=pltpu.MemorySpace.HBM),
        ],
        out_specs=pl.BlockSpec(memory_space=pltpu.VMEM),
        scratch_shapes=[
            pltpu.VMEM((2, k, n_per), w_mat.dtype),
            pltpu.VMEM((2, m_per, n_per), jnp.float32),
            pltpu.SemaphoreType.DMA((2,)),
            pltpu.SemaphoreType.DMA((2,)),
            pltpu.SemaphoreType.DMA((N_DEV,)),
        ],
        compiler_params=pltpu.CompilerParams(
            vmem_limit_bytes=100 * 1024 * 1024,
        ),
    )(x, w_mat)
